# baseline (speedup 1.0000x reference)
import functools

import jax
import jax.numpy as jnp
from jax import lax
from jax.experimental import pallas as pl
from jax.experimental.pallas import tpu as pltpu
from jax.experimental.pallas import tpu_sc as plsc

NUM_EMB = 1_000_000
D = 64
L = 16
NC = 2
NS = 16
NW = NC * NS
B1 = 16384
B2 = 50
B = B1 * B2
B_PER_W = B // NW
CHUNK = 128
N_CHUNK = B_PER_W // CHUNK
CHUNKS_PER_B2 = B1 // CHUNK
NBUF = 2
KB = 4
N_GRP = N_CHUNK // KB
PITCH = CHUNK + 1
_MAGIC = 0x5F3759DF


def _rsqrt16(x):
    i = plsc.bitcast(x, jnp.int32)
    i = jnp.int32(_MAGIC) - lax.shift_right_arithmetic(i, jnp.int32(1))
    y = plsc.bitcast(i, jnp.float32)
    for _ in range(3):
        y = y * (1.5 - 0.5 * x * y * y)
    return y


def _hsum_all(x):
    dnums = lax.GatherDimensionNumbers(
        offset_dims=(), collapsed_slice_dims=(0,), start_index_map=(0,))
    for k in (1, 2, 4, 8):
        perm = lax.iota(jnp.int32, L) ^ k
        x = x + lax.gather(x, perm[:, None], dnums, slice_sizes=(1,),
                           mode=lax.GatherScatterMode.PROMISE_IN_BOUNDS)
    return x


def _sc_kernel(idx_hbm, table_hbm, out_hbm, idx_v, gbuf, tbuf, sbuf,
               gsems, ssem):
    wid = lax.axis_index("s") * NC + lax.axis_index("c")

    pltpu.sync_copy(idx_hbm.at[wid], idx_v)

    def start_gather(b, c):
        pltpu.async_copy(table_hbm.at[idx_v.at[c]], gbuf.at[b], gsems[b])

    def wait_gather(b, c):
        pltpu.make_async_copy(table_hbm.at[idx_v.at[c]], gbuf.at[b],
                              gsems[b]).wait()

    def store_refs(grp, jb):
        gc = wid * N_CHUNK + grp * KB
        b2 = gc // CHUNKS_PER_B2
        blk = gc % CHUNKS_PER_B2
        return sbuf.at[jb], out_hbm.at[b2, jb, pl.ds(blk, KB)]

    def start_store(grp):
        for jb in range(D // 8):
            src, dst = store_refs(grp, jb)
            pltpu.async_copy(src, dst, ssem)

    def wait_store(grp):
        for jb in range(D // 8):
            src, dst = store_refs(grp, jb)
            pltpu.make_async_copy(src, dst, ssem).wait()

    for b in range(NBUF):
        start_gather(b, b)

    iota = lax.iota(jnp.int32, L)

    def grp_body(grp, carry):
        @pl.when(grp >= 1)
        def _():
            wait_store(grp - 1)

        for k in range(KB):
            c = grp * KB + k
            b = k % NBUF
            wait_gather(b, c)

            def row_body(r, carry2):
                v0 = gbuf[b, r, pl.ds(0, L)]
                v1 = gbuf[b, r, pl.ds(L, L)]
                v2 = gbuf[b, r, pl.ds(2 * L, L)]
                v3 = gbuf[b, r, pl.ds(3 * L, L)]
                ss = v0 * v0 + v1 * v1 + v2 * v2 + v3 * v3
                m = _hsum_all(ss) * (1.0 / D) + 1e-30
                y = _rsqrt16(m)
                rr = jnp.full((L,), r, dtype=jnp.int32)
                for q, v in enumerate((v0, v1, v2, v3)):
                    plsc.store_scatter(tbuf, [q * L + iota, rr], v * y)
                return carry2

            lax.fori_loop(0, CHUNK, row_body, 0)

            @pl.when(c + NBUF < N_CHUNK)
            def _():
                start_gather(b, c + NBUF)

            def pack_body(j, carry3):
                jb = j // 8
                js = j % 8
                for g in range(CHUNK // L):
                    sbuf[jb, k, js, pl.ds(g * L, L)] = (
                        tbuf[j, pl.ds(g * L, L)])
                return carry3

            lax.fori_loop(0, D, pack_body, 0)

        start_store(grp)
        return carry

    lax.fori_loop(0, N_GRP, grp_body, 0)
    wait_store(N_GRP - 1)


@jax.jit
def _run(idx, table):
    mesh = plsc.VectorSubcoreMesh(core_axis_name="c", subcore_axis_name="s")
    f = functools.partial(
        pl.kernel,
        mesh=mesh,
        compiler_params=pltpu.CompilerParams(needs_layout_passes=False,
                                             use_tc_tiling_on_sc=False),
        out_type=jax.ShapeDtypeStruct((B2, 8, CHUNKS_PER_B2, 8, CHUNK),
                                      jnp.float32),
        scratch_types=[
            pltpu.VMEM((N_CHUNK, CHUNK), jnp.int32),
            pltpu.VMEM((NBUF, CHUNK, D), jnp.float32),
            pltpu.VMEM((D, PITCH), jnp.float32),
            pltpu.VMEM((8, KB, 8, CHUNK), jnp.float32),
            [pltpu.SemaphoreType.DMA] * NBUF,
            pltpu.SemaphoreType.DMA,
        ],
    )(_sc_kernel)
    return f(idx, table)


def kernel(input, raw_weight):
    idx = input.T.reshape(NW, N_CHUNK, CHUNK).astype(jnp.int32)
    out5 = _run(idx, raw_weight)
    out = out5.transpose(0, 1, 3, 2, 4).reshape(B2, D, B1)
    return out.transpose(2, 0, 1)


# speedup vs baseline: 2.0816x; 2.0816x over previous
import functools

import jax
import jax.numpy as jnp
from jax import lax
from jax.experimental import pallas as pl
from jax.experimental.pallas import tpu as pltpu
from jax.experimental.pallas import tpu_sc as plsc

NUM_EMB = 1_000_000
D = 64
L = 16
NC = 2
NS = 16
NW = NC * NS
B1 = 16384
B2 = 50
B = B1 * B2
B_PER_W = B // NW
CHUNK = 128
N_CHUNK = B_PER_W // CHUNK
CHUNKS_PER_B2 = B1 // CHUNK
NBUF = 2
KB = 4
N_GRP = N_CHUNK // KB
PITCH = CHUNK + 1
_MAGIC = 0x5F3759DF


def _rsqrt16(x):
    i = plsc.bitcast(x, jnp.int32)
    i = jnp.int32(_MAGIC) - lax.shift_right_arithmetic(i, jnp.int32(1))
    y = plsc.bitcast(i, jnp.float32)
    for _ in range(3):
        y = y * (1.5 - 0.5 * x * y * y)
    return y


def _hsum_all(x):
    dnums = lax.GatherDimensionNumbers(
        offset_dims=(), collapsed_slice_dims=(0,), start_index_map=(0,))
    for k in (1, 2, 4, 8):
        perm = lax.iota(jnp.int32, L) ^ k
        x = x + lax.gather(x, perm[:, None], dnums, slice_sizes=(1,),
                           mode=lax.GatherScatterMode.PROMISE_IN_BOUNDS)
    return x


def _sc_kernel(idx_hbm, table_hbm, out_hbm, idx_v, gbuf, tbuf, sbuf,
               gsems, ssem):
    wid = lax.axis_index("s") * NC + lax.axis_index("c")

    pltpu.sync_copy(idx_hbm.at[wid], idx_v)

    def start_gather(b, c):
        pltpu.async_copy(table_hbm.at[idx_v.at[c]], gbuf.at[b], gsems[b])

    def wait_gather(b, c):
        pltpu.make_async_copy(table_hbm.at[idx_v.at[c]], gbuf.at[b],
                              gsems[b]).wait()

    def store_refs(grp, jb):
        gc = wid * N_CHUNK + grp * KB
        b2 = gc // CHUNKS_PER_B2
        blk = gc % CHUNKS_PER_B2
        return sbuf.at[jb], out_hbm.at[b2, jb, pl.ds(blk, KB)]

    def start_store(grp):
        for jb in range(D // 8):
            src, dst = store_refs(grp, jb)
            pltpu.async_copy(src, dst, ssem)

    def wait_store(grp):
        for jb in range(D // 8):
            src, dst = store_refs(grp, jb)
            pltpu.make_async_copy(src, dst, ssem).wait()

    for b in range(NBUF):
        start_gather(b, b)

    iota = lax.iota(jnp.int32, L)

    def grp_body(grp, carry):
        @pl.when(grp >= 1)
        def _():
            wait_store(grp - 1)

        for k in range(KB):
            c = grp * KB + k
            b = k % NBUF
            wait_gather(b, c)

            @plsc.parallel_loop(0, CHUNK, unroll=4)
            def row_body(r):
                v0 = gbuf[b, r, pl.ds(0, L)]
                v1 = gbuf[b, r, pl.ds(L, L)]
                v2 = gbuf[b, r, pl.ds(2 * L, L)]
                v3 = gbuf[b, r, pl.ds(3 * L, L)]
                ss = v0 * v0 + v1 * v1 + v2 * v2 + v3 * v3
                m = _hsum_all(ss) * (1.0 / D) + 1e-30
                y = _rsqrt16(m)
                rr = jnp.full((L,), r, dtype=jnp.int32)
                for q, v in enumerate((v0, v1, v2, v3)):
                    plsc.store_scatter(tbuf, [q * L + iota, rr], v * y)

            @pl.when(c + NBUF < N_CHUNK)
            def _():
                start_gather(b, c + NBUF)

            @plsc.parallel_loop(0, D, unroll=4)
            def pack_body(j):
                jb = j // 8
                js = j % 8
                for g in range(CHUNK // L):
                    sbuf[jb, k, js, pl.ds(g * L, L)] = (
                        tbuf[j, pl.ds(g * L, L)])

        start_store(grp)
        return carry

    lax.fori_loop(0, N_GRP, grp_body, 0)
    wait_store(N_GRP - 1)


@jax.jit
def _run(idx, table):
    mesh = plsc.VectorSubcoreMesh(core_axis_name="c", subcore_axis_name="s")
    f = functools.partial(
        pl.kernel,
        mesh=mesh,
        compiler_params=pltpu.CompilerParams(needs_layout_passes=False,
                                             use_tc_tiling_on_sc=False),
        out_type=jax.ShapeDtypeStruct((B2, 8, CHUNKS_PER_B2, 8, CHUNK),
                                      jnp.float32),
        scratch_types=[
            pltpu.VMEM((N_CHUNK, CHUNK), jnp.int32),
            pltpu.VMEM((NBUF, CHUNK, D), jnp.float32),
            pltpu.VMEM((D, PITCH), jnp.float32),
            pltpu.VMEM((8, KB, 8, CHUNK), jnp.float32),
            [pltpu.SemaphoreType.DMA] * NBUF,
            pltpu.SemaphoreType.DMA,
        ],
    )(_sc_kernel)
    return f(idx, table)


def kernel(input, raw_weight):
    idx = input.T.reshape(NW, N_CHUNK, CHUNK).astype(jnp.int32)
    out5 = _run(idx, raw_weight)
    out = out5.transpose(0, 1, 3, 2, 4).reshape(B2, D, B1)
    return out.transpose(2, 0, 1)
